# baseline (speedup 1.0000x reference)
import functools

import jax
import jax.numpy as jnp
from jax import lax
from jax.experimental import pallas as pl
from jax.experimental.pallas import tpu as pltpu
from jax.experimental.pallas import tpu_sc as plsc

NC = 2
NS = 16
NW = NC * NS
LANES = 16
CHUNK = 128
SUP = 16
G = 128


def _sc_mesh():
    return plsc.VectorSubcoreMesh(
        core_axis_name="c", subcore_axis_name="s", num_cores=NC, num_subcores=NS
    )


_SC_PARAMS = pltpu.CompilerParams(
    use_tc_tiling_on_sc=False, needs_layout_passes=False
)


def _fill_f32(ref, rows, value):
    def body(i, _):
        ref[i] = jnp.full((LANES,), value, jnp.float32)
        return 0

    lax.fori_loop(0, rows, body, 0)


def _set_idx_row(ref, row, value):
    v = jnp.full((LANES,), value, jnp.int32)
    for q in range(CHUNK // LANES):
        ref[row, pl.ds(q * LANES, LANES)] = v


def _zero_spmem(zero_v, zrows, sh, tbase, rt):
    for p in range(rt // zrows):
        pltpu.sync_copy(zero_v, sh.at[pl.ds(tbase + p * zrows, zrows)])


def _load_edges(edge_hbm, which, base, full, rem, idx_v, mini_v, sem):
    def row(q, _):
        pltpu.async_copy(
            edge_hbm.at[which, pl.ds(base + q * CHUNK, CHUNK)], idx_v.at[q], sem
        )
        return 0

    lax.fori_loop(0, full, row, 0)
    if rem:
        pltpu.async_copy(
            edge_hbm.at[which, pl.ds(base + full * CHUNK, rem)],
            mini_v.at[0], sem,
        )

    def drain(q, _):
        pltpu.make_async_copy(
            edge_hbm.at[which, pl.ds(base, CHUNK)], idx_v.at[0], sem
        ).wait()
        return 0

    lax.fori_loop(0, full, drain, 0)
    if rem:
        pltpu.make_async_copy(
            edge_hbm.at[which, pl.ds(base, rem)], mini_v.at[0], sem
        ).wait()


HR = 128


def _make_hist_kernel(n, nr, ept):
    full = ept // CHUNK
    rem = ept - full * CHUNK
    rpt = HR // NS

    @functools.partial(
        pl.kernel,
        out_type=(
            jax.ShapeDtypeStruct((NC, HR, G), jnp.float32),
            jax.ShapeDtypeStruct((NC, HR, G), jnp.float32),
        ),
        mesh=_sc_mesh(),
        compiler_params=_SC_PARAMS,
        scratch_types=[
            pltpu.VMEM((full, CHUNK), jnp.int32),
            pltpu.VMEM((full, CHUNK), jnp.int32),
            pltpu.VMEM((1, LANES), jnp.int32),
            pltpu.VMEM((1, LANES), jnp.int32),
            pltpu.VMEM((HR, G), jnp.float32),
            pltpu.VMEM((HR, G), jnp.float32),
            pltpu.VMEM((1, G), jnp.int32),
            pltpu.VMEM((rpt, G), jnp.float32),
            pltpu.VMEM_SHARED((HR, G), jnp.float32),
            pltpu.VMEM_SHARED((HR, G), jnp.float32),
            pltpu.SemaphoreType.DMA,
            pltpu.SemaphoreType.DMA,
            pltpu.SemaphoreType.DMA,
        ],
    )
    def hist_kernel(edge_hbm, out0_hbm, out1_hbm,
                    src_v, dst_v, sx_v, dx_v, hl0, hl1, midx, zero_v,
                    h0_sh, h1_sh, lsem, semS, semD):
        c = lax.axis_index("c")
        s = lax.axis_index("s")
        wid = c * NS + s
        base = wid * ept
        tbase = s * rpt

        zrow = jnp.zeros((LANES,), jnp.float32)

        def zl(i, _):
            r = i // (G // LANES)
            q = i % (G // LANES)
            hl0[r, pl.ds(q * LANES, LANES)] = zrow
            hl1[r, pl.ds(q * LANES, LANES)] = zrow
            return 0

        lax.fori_loop(0, HR * (G // LANES), zl, 0)
        for i in range(rpt):
            for q in range(G // LANES):
                zero_v[i, pl.ds(q * LANES, LANES)] = zrow
        for k in range(G // LANES):
            midx[0, pl.ds(k * LANES, LANES)] = (
                lax.iota(jnp.int32, LANES) + k * LANES
            )
        pltpu.sync_copy(zero_v, h0_sh.at[pl.ds(tbase, rpt)])
        pltpu.sync_copy(zero_v, h1_sh.at[pl.ds(tbase, rpt)])
        _load_edges(edge_hbm, 0, base, full, rem, src_v, sx_v, lsem)
        _load_edges(edge_hbm, 1, base, full, rem, dst_v, dx_v, lsem)
        if rem:
            v = jnp.full((LANES,), n, jnp.int32)
            lanes = lax.broadcasted_iota(jnp.int32, (LANES,), 0)
            sx_v[0] = jnp.where(lanes < rem, sx_v[0], v)
            dx_v[0] = jnp.where(lanes < rem, dx_v[0], v)

        ones = jnp.ones((LANES,), jnp.float32)

        def chunk(j, _):
            for q in range(CHUNK // LANES):
                i0 = src_v[j, pl.ds(q * LANES, LANES)]
                plsc.addupdate_scatter(
                    hl0, [lax.shift_right_logical(i0, 7), i0 & 127], ones
                )
                i1 = dst_v[j, pl.ds(q * LANES, LANES)]
                plsc.addupdate_scatter(
                    hl1, [lax.shift_right_logical(i1, 7), i1 & 127], ones
                )
            return 0

        lax.fori_loop(0, full, chunk, 0)
        if rem:
            i0 = sx_v[0]
            plsc.addupdate_scatter(
                hl0, [lax.shift_right_logical(i0, 7), i0 & 127], ones
            )
            i1 = dx_v[0]
            plsc.addupdate_scatter(
                hl1, [lax.shift_right_logical(i1, 7), i1 & 127], ones
            )
        plsc.subcore_barrier()

        pltpu.async_copy(hl0, h0_sh.at[midx.at[0]], semS, add=True)
        pltpu.async_copy(hl1, h1_sh.at[midx.at[0]], semD, add=True)
        pltpu.make_async_copy(hl0, h0_sh.at[midx.at[0]], semS).wait()
        pltpu.make_async_copy(hl1, h1_sh.at[midx.at[0]], semD).wait()
        plsc.subcore_barrier()

        pltpu.sync_copy(h0_sh.at[pl.ds(tbase, rpt)], out0_hbm.at[c, pl.ds(tbase, rpt)])
        pltpu.sync_copy(h1_sh.at[pl.ds(tbase, rpt)], out1_hbm.at[c, pl.ds(tbase, rpt)])

    return hist_kernel


def _make_scatter_kernel(n, nr, ept):
    rt = nr // NS
    full = ept // CHUNK
    rem = ept - full * CHUNK
    full_s = ((full + SUP - 1) // SUP) * SUP
    nsup = full_s // SUP
    zrows = rt // 4

    @functools.partial(
        pl.kernel,
        out_type=jax.ShapeDtypeStruct((NC, nr, LANES), jnp.float32),
        mesh=_sc_mesh(),
        compiler_params=_SC_PARAMS,
        scratch_types=[
            pltpu.VMEM((full_s, CHUNK), jnp.int32),
            pltpu.VMEM((full_s, CHUNK), jnp.int32),
            pltpu.VMEM((1, LANES), jnp.int32),
            pltpu.VMEM((1, LANES), jnp.int32),
            [pltpu.VMEM((SUP * CHUNK, LANES), jnp.float32) for _ in range(2)],
            pltpu.VMEM((LANES, LANES), jnp.float32),
            pltpu.VMEM((zrows, LANES), jnp.float32),
            pltpu.VMEM_SHARED((nr, LANES), jnp.float32),
            pltpu.SemaphoreType.DMA,
            [pltpu.SemaphoreType.DMA for _ in range(2)],
            [pltpu.SemaphoreType.DMA for _ in range(2)],
        ],
    )
    def scatter_kernel(h_hbm, edge_hbm, out_hbm,
                       src_v, dst_v, sx_v, dx_v, bufs, rmini_v, zero_v,
                       agg_sh, lsem, gsems, ssems):
        c = lax.axis_index("c")
        s = lax.axis_index("s")
        wid = c * NS + s
        base = wid * ept
        tbase = s * rt

        _fill_f32(zero_v, zrows, 0.0)
        for r in range(full, full_s):
            _set_idx_row(src_v, r, 0)
            _set_idx_row(dst_v, r, n)
        _zero_spmem(zero_v, zrows, agg_sh, tbase, rt)
        _load_edges(edge_hbm, 0, base, full, rem, src_v, sx_v, lsem)
        _load_edges(edge_hbm, 1, base, full, rem, dst_v, dx_v, lsem)
        if rem:
            v0 = jnp.zeros((LANES,), jnp.int32)
            vn = jnp.full((LANES,), n, jnp.int32)
            lanes = lax.broadcasted_iota(jnp.int32, (LANES,), 0)
            sx_v[0] = jnp.where(lanes < rem, sx_v[0], v0)
            dx_v[0] = jnp.where(lanes < rem, dx_v[0], vn)
        plsc.subcore_barrier()

        def gathers(t, buf, gsem):
            def g(k, _):
                pltpu.async_copy(
                    h_hbm.at[src_v.at[t * SUP + k]],
                    buf.at[pl.ds(k * CHUNK, CHUNK)], gsem,
                )
                return 0
            lax.fori_loop(0, SUP, g, 0)

        def process(t, buf, gsem, ssem):
            def sc(k, _):
                pltpu.make_async_copy(
                    h_hbm.at[src_v.at[0]], buf.at[pl.ds(0, CHUNK)], gsem
                ).wait()
                pltpu.async_copy(
                    buf.at[pl.ds(k * CHUNK, CHUNK)],
                    agg_sh.at[dst_v.at[t * SUP + k]], ssem, add=True,
                )
                return 0
            lax.fori_loop(0, SUP, sc, 0)

        def drain_scatters(buf, ssem):
            def d(k, _):
                pltpu.make_async_copy(
                    buf.at[pl.ds(0, CHUNK)], agg_sh.at[dst_v.at[0]], ssem
                ).wait()
                return 0
            lax.fori_loop(0, SUP, d, 0)

        gathers(0, bufs[0], gsems[0])
        for t in range(nsup):
            p = t % 2
            o = 1 - p
            if t + 1 < nsup:
                if t >= 1:
                    drain_scatters(bufs[o], ssems[o])
                gathers(t + 1, bufs[o], gsems[o])
            process(t, bufs[p], gsems[p], ssems[p])
        drain_scatters(bufs[(nsup - 1) % 2], ssems[(nsup - 1) % 2])
        if nsup >= 2:
            drain_scatters(bufs[nsup % 2], ssems[nsup % 2])

        if rem:
            pltpu.sync_copy(h_hbm.at[sx_v.at[0]], rmini_v)
            pltpu.sync_copy(rmini_v, agg_sh.at[dx_v.at[0]], add=True)

        plsc.subcore_barrier()
        pltpu.sync_copy(agg_sh.at[pl.ds(tbase, rt)], out_hbm.at[c, pl.ds(tbase, rt)])

    return scatter_kernel


def _matmul_body(x_ref, w_ref, o_ref):
    o_ref[...] = jnp.dot(x_ref[...], w_ref[...],
                         preferred_element_type=jnp.float32)


def _expand8(norm8):
    la = lax.broadcasted_iota(jnp.int32, (8, G), 1) // LANES
    ra = lax.broadcasted_iota(jnp.int32, (8, G), 0)
    exp = jnp.where(la == ra, 1.0, 0.0)
    return lax.dot_general(
        norm8, exp, dimension_numbers=(((1,), (0,)), ((), ())),
        preferred_element_type=jnp.float32,
    )


def _scale_body(np8, deg_ref, h_ref, o_ref):
    d = deg_ref[0] + deg_ref[1]
    d = lax.slice(d, (0, 0), (np8, 8))
    norm8 = lax.rsqrt(jnp.maximum(d, 1.0))
    o_ref[...] = h_ref[...] * _expand8(norm8)


def _readout_body(n_valid, np8, agg_ref, deg_ref, ids_ref, b_ref, o_ref):
    a = agg_ref[0] + agg_ref[1]
    d = deg_ref[0] + deg_ref[1]
    d = lax.slice(d, (0, 0), (np8, 8))
    norm8 = lax.rsqrt(jnp.maximum(d, 1.0))
    norm = _expand8(norm8)
    lane = lax.broadcasted_iota(jnp.int32, (np8, G), 1)
    prow = lax.broadcasted_iota(jnp.int32, (np8, G), 0)
    node = prow * 8 + lane // LANES
    hn = a * norm + b_ref[...]
    hn = hn + jnp.where(lane % LANES == 10, 1.0, 0.0)
    hn = jnp.where(node < n_valid, hn, 0.0)
    gids = lax.broadcasted_iota(jnp.int32, (np8, G), 1)
    acc = jnp.zeros((G, LANES), jnp.float32)
    for a8 in range(8):
        hn_a = lax.slice(hn, (0, a8 * LANES), (np8, (a8 + 1) * LANES))
        ids_a = ids_ref[a8]
        oh = jnp.where(ids_a[:, None] == gids, 1.0, 0.0)
        acc = acc + lax.dot_general(
            oh, hn_a, dimension_numbers=(((0,), (0,)), ((), ())),
            preferred_element_type=jnp.float32,
        )
    glane = lax.broadcasted_iota(jnp.int32, (G, LANES), 1)
    cnt = jnp.sum(jnp.where(glane == 10, acc, 0.0), axis=1, keepdims=True)
    o_ref[...] = acc / jnp.maximum(cnt, 1.0)


def kernel(in_feat, edge_index, node_graph_ids, W, b):
    n, d_in = in_feat.shape
    e = edge_index.shape[1]
    c_out = W.shape[1]

    nr = ((n + 1 + 127) // 128) * 128
    ept = e // NW
    np8 = n // 8
    nr8 = nr // 8

    w_pad = jnp.pad(W, ((0, 0), (0, LANES - c_out)))
    w_big = (jnp.eye(8, dtype=jnp.float32)[:, None, :, None]
             * w_pad[None, :, None, :]).reshape(8 * d_in, G)
    b_tile = jnp.tile(jnp.pad(b, (0, LANES - c_out)), 8).reshape(1, G)
    ids_pad = jnp.pad(node_graph_ids, (0, nr - n))
    ids_strided = ids_pad.reshape(nr8, 8).T

    deg_out, deg_in = _make_hist_kernel(n, nr, ept)(edge_index)
    deg_out_p = deg_out.reshape(NC, HR * G // 8, 8)
    deg_in_p = deg_in.reshape(NC, HR * G // 8, 8)

    x_packed = in_feat.reshape(np8, 8 * d_in)
    h_packed = pl.pallas_call(
        _matmul_body,
        grid=(1,),
        in_specs=[
            pl.BlockSpec((np8, 8 * d_in), lambda i: (0, 0)),
            pl.BlockSpec((8 * d_in, G), lambda i: (0, 0)),
        ],
        out_specs=pl.BlockSpec((np8, G), lambda i: (0, 0)),
        out_shape=jax.ShapeDtypeStruct((np8, G), jnp.float32),
    )(x_packed, w_big)

    h_scaled = pl.pallas_call(
        functools.partial(_scale_body, np8),
        grid=(1,),
        in_specs=[
            pl.BlockSpec((NC, HR * G // 8, 8), lambda i: (0, 0, 0)),
            pl.BlockSpec((np8, G), lambda i: (0, 0)),
        ],
        out_specs=pl.BlockSpec((np8, G), lambda i: (0, 0)),
        out_shape=jax.ShapeDtypeStruct((np8, G), jnp.float32),
    )(deg_out_p, h_packed)

    agg = _make_scatter_kernel(n, nr, ept)(
        h_scaled.reshape(n, LANES), edge_index
    )
    agg_p = agg.reshape(NC, nr8, G)

    out16 = pl.pallas_call(
        functools.partial(_readout_body, n, nr8),
        grid=(1,),
        in_specs=[
            pl.BlockSpec((NC, nr8, G), lambda i: (0, 0, 0)),
            pl.BlockSpec((NC, HR * G // 8, 8), lambda i: (0, 0, 0)),
            pl.BlockSpec((8, nr8), lambda i: (0, 0)),
            pl.BlockSpec((1, G), lambda i: (0, 0)),
        ],
        out_specs=pl.BlockSpec((G, LANES), lambda i: (0, 0)),
        out_shape=jax.ShapeDtypeStruct((G, LANES), jnp.float32),
    )(agg_p, deg_in_p, ids_strided, b_tile)

    return out16[:, :c_out]


# speedup vs baseline: 1.0028x; 1.0028x over previous
import functools

import jax
import jax.numpy as jnp
from jax import lax
from jax.experimental import pallas as pl
from jax.experimental.pallas import tpu as pltpu
from jax.experimental.pallas import tpu_sc as plsc

NC = 2
NS = 16
NW = NC * NS
LANES = 16
CHUNK = 128
SUP = 16
G = 128


def _sc_mesh():
    return plsc.VectorSubcoreMesh(
        core_axis_name="c", subcore_axis_name="s", num_cores=NC, num_subcores=NS
    )


_SC_PARAMS = pltpu.CompilerParams(use_tc_tiling_on_sc=False)


def _fill_f32(ref, rows, value):
    def body(i, _):
        ref[i] = jnp.full((LANES,), value, jnp.float32)
        return 0

    lax.fori_loop(0, rows, body, 0)


def _set_idx_row(ref, row, value):
    v = jnp.full((LANES,), value, jnp.int32)
    for q in range(CHUNK // LANES):
        ref[row, pl.ds(q * LANES, LANES)] = v


def _zero_spmem(zero_v, zrows, sh, tbase, rt):
    for p in range(rt // zrows):
        pltpu.sync_copy(zero_v, sh.at[pl.ds(tbase + p * zrows, zrows)])


def _load_edges(edge_hbm, which, base, full, rem, idx_v, mini_v, sem):
    def row(q, _):
        pltpu.async_copy(
            edge_hbm.at[which, pl.ds(base + q * CHUNK, CHUNK)], idx_v.at[q], sem
        )
        return 0

    lax.fori_loop(0, full, row, 0)
    if rem:
        pltpu.async_copy(
            edge_hbm.at[which, pl.ds(base + full * CHUNK, rem)],
            mini_v.at[0], sem,
        )

    def drain(q, _):
        pltpu.make_async_copy(
            edge_hbm.at[which, pl.ds(base, CHUNK)], idx_v.at[0], sem
        ).wait()
        return 0

    lax.fori_loop(0, full, drain, 0)
    if rem:
        pltpu.make_async_copy(
            edge_hbm.at[which, pl.ds(base, rem)], mini_v.at[0], sem
        ).wait()


def _make_hist_kernel(n, nr, ept):
    rt = nr // NS
    full = ept // CHUNK
    rem = ept - full * CHUNK
    zrows = rt // 4

    @functools.partial(
        pl.kernel,
        out_type=(
            jax.ShapeDtypeStruct((NC, nr, LANES), jnp.float32),
            jax.ShapeDtypeStruct((NC, nr, LANES), jnp.float32),
        ),
        mesh=_sc_mesh(),
        compiler_params=_SC_PARAMS,
        scratch_types=[
            pltpu.VMEM((full, CHUNK), jnp.int32),
            pltpu.VMEM((full, CHUNK), jnp.int32),
            pltpu.VMEM((1, LANES), jnp.int32),
            pltpu.VMEM((1, LANES), jnp.int32),
            pltpu.VMEM((CHUNK, LANES), jnp.float32),
            pltpu.VMEM((zrows, LANES), jnp.float32),
            pltpu.VMEM_SHARED((nr, LANES), jnp.float32),
            pltpu.VMEM_SHARED((nr, LANES), jnp.float32),
            pltpu.SemaphoreType.DMA,
            pltpu.SemaphoreType.DMA,
            pltpu.SemaphoreType.DMA,
        ],
    )
    def hist_kernel(edge_hbm, out0_hbm, out1_hbm,
                    src_v, dst_v, sx_v, dx_v, ones_v, zero_v,
                    h0_sh, h1_sh, lsem, semS, semD):
        c = lax.axis_index("c")
        s = lax.axis_index("s")
        wid = c * NS + s
        base = wid * ept
        tbase = s * rt

        _fill_f32(ones_v, CHUNK, 1.0)
        _fill_f32(zero_v, zrows, 0.0)
        _zero_spmem(zero_v, zrows, h0_sh, tbase, rt)
        _zero_spmem(zero_v, zrows, h1_sh, tbase, rt)
        _load_edges(edge_hbm, 0, base, full, rem, src_v, sx_v, lsem)
        _load_edges(edge_hbm, 1, base, full, rem, dst_v, dx_v, lsem)
        if rem:
            v = jnp.full((LANES,), n, jnp.int32)
            lanes = lax.broadcasted_iota(jnp.int32, (LANES,), 0)
            sx_v[0] = jnp.where(lanes < rem, sx_v[0], v)
            dx_v[0] = jnp.where(lanes < rem, dx_v[0], v)
        plsc.subcore_barrier()

        def chunk(j, _):
            pltpu.async_copy(ones_v, h0_sh.at[src_v.at[j]], semS, add=True)
            pltpu.async_copy(ones_v, h1_sh.at[dst_v.at[j]], semD, add=True)
            return 0

        lax.fori_loop(0, full, chunk, 0)
        if rem:
            pltpu.async_copy(
                ones_v.at[pl.ds(0, LANES)], h0_sh.at[sx_v.at[0]], semS, add=True
            )
            pltpu.async_copy(
                ones_v.at[pl.ds(0, LANES)], h1_sh.at[dx_v.at[0]], semD, add=True
            )

        def drain(j, _):
            pltpu.make_async_copy(ones_v, h0_sh.at[src_v.at[0]], semS).wait()
            pltpu.make_async_copy(ones_v, h1_sh.at[dst_v.at[0]], semD).wait()
            return 0

        lax.fori_loop(0, full, drain, 0)
        if rem:
            pltpu.make_async_copy(
                ones_v.at[pl.ds(0, LANES)], h0_sh.at[sx_v.at[0]], semS
            ).wait()
            pltpu.make_async_copy(
                ones_v.at[pl.ds(0, LANES)], h1_sh.at[dx_v.at[0]], semD
            ).wait()
        plsc.subcore_barrier()

        pltpu.sync_copy(h0_sh.at[pl.ds(tbase, rt)], out0_hbm.at[c, pl.ds(tbase, rt)])
        pltpu.sync_copy(h1_sh.at[pl.ds(tbase, rt)], out1_hbm.at[c, pl.ds(tbase, rt)])

    return hist_kernel


def _make_scatter_kernel(n, nr, ept):
    rt = nr // NS
    full = ept // CHUNK
    rem = ept - full * CHUNK
    full_s = ((full + SUP - 1) // SUP) * SUP
    nsup = full_s // SUP
    zrows = rt // 4

    @functools.partial(
        pl.kernel,
        out_type=jax.ShapeDtypeStruct((NC, nr, LANES), jnp.float32),
        mesh=_sc_mesh(),
        compiler_params=_SC_PARAMS,
        scratch_types=[
            pltpu.VMEM((full_s, CHUNK), jnp.int32),
            pltpu.VMEM((full_s, CHUNK), jnp.int32),
            pltpu.VMEM((1, LANES), jnp.int32),
            pltpu.VMEM((1, LANES), jnp.int32),
            [pltpu.VMEM((SUP * CHUNK, LANES), jnp.float32) for _ in range(2)],
            pltpu.VMEM((LANES, LANES), jnp.float32),
            pltpu.VMEM((zrows, LANES), jnp.float32),
            pltpu.VMEM_SHARED((nr, LANES), jnp.float32),
            pltpu.SemaphoreType.DMA,
            [pltpu.SemaphoreType.DMA for _ in range(2)],
            [pltpu.SemaphoreType.DMA for _ in range(2)],
        ],
    )
    def scatter_kernel(h_hbm, edge_hbm, out_hbm,
                       src_v, dst_v, sx_v, dx_v, bufs, rmini_v, zero_v,
                       agg_sh, lsem, gsems, ssems):
        c = lax.axis_index("c")
        s = lax.axis_index("s")
        wid = c * NS + s
        base = wid * ept
        tbase = s * rt

        _fill_f32(zero_v, zrows, 0.0)
        for r in range(full, full_s):
            _set_idx_row(src_v, r, 0)
            _set_idx_row(dst_v, r, n)
        _zero_spmem(zero_v, zrows, agg_sh, tbase, rt)
        _load_edges(edge_hbm, 0, base, full, rem, src_v, sx_v, lsem)
        _load_edges(edge_hbm, 1, base, full, rem, dst_v, dx_v, lsem)
        if rem:
            v0 = jnp.zeros((LANES,), jnp.int32)
            vn = jnp.full((LANES,), n, jnp.int32)
            lanes = lax.broadcasted_iota(jnp.int32, (LANES,), 0)
            sx_v[0] = jnp.where(lanes < rem, sx_v[0], v0)
            dx_v[0] = jnp.where(lanes < rem, dx_v[0], vn)
        plsc.subcore_barrier()

        def gathers(t, buf, gsem):
            def g(k, _):
                pltpu.async_copy(
                    h_hbm.at[src_v.at[t * SUP + k]],
                    buf.at[pl.ds(k * CHUNK, CHUNK)], gsem,
                )
                return 0
            lax.fori_loop(0, SUP, g, 0)

        def process(t, buf, gsem, ssem):
            def sc(k, _):
                pltpu.make_async_copy(
                    h_hbm.at[src_v.at[0]], buf.at[pl.ds(0, CHUNK)], gsem
                ).wait()
                pltpu.async_copy(
                    buf.at[pl.ds(k * CHUNK, CHUNK)],
                    agg_sh.at[dst_v.at[t * SUP + k]], ssem, add=True,
                )
                return 0
            lax.fori_loop(0, SUP, sc, 0)

        def drain_scatters(buf, ssem):
            def d(k, _):
                pltpu.make_async_copy(
                    buf.at[pl.ds(0, CHUNK)], agg_sh.at[dst_v.at[0]], ssem
                ).wait()
                return 0
            lax.fori_loop(0, SUP, d, 0)

        gathers(0, bufs[0], gsems[0])
        for t in range(nsup):
            p = t % 2
            o = 1 - p
            if t + 1 < nsup:
                if t >= 1:
                    drain_scatters(bufs[o], ssems[o])
                gathers(t + 1, bufs[o], gsems[o])
            process(t, bufs[p], gsems[p], ssems[p])
        drain_scatters(bufs[(nsup - 1) % 2], ssems[(nsup - 1) % 2])
        if nsup >= 2:
            drain_scatters(bufs[nsup % 2], ssems[nsup % 2])

        if rem:
            pltpu.sync_copy(h_hbm.at[sx_v.at[0]], rmini_v)
            pltpu.sync_copy(rmini_v, agg_sh.at[dx_v.at[0]], add=True)

        plsc.subcore_barrier()
        pltpu.sync_copy(agg_sh.at[pl.ds(tbase, rt)], out_hbm.at[c, pl.ds(tbase, rt)])

    return scatter_kernel


def _matmul_body(x_ref, w_ref, o_ref):
    o_ref[...] = jnp.dot(x_ref[...], w_ref[...],
                         preferred_element_type=jnp.float32)


def _scale_body(np8, deg_ref, h_ref, o_ref):
    d = deg_ref[0] + deg_ref[1]
    d = lax.slice(d, (0, 0), (np8, G))
    norm = lax.rsqrt(jnp.maximum(d, 1.0))
    o_ref[...] = h_ref[...] * norm


def _readout_body(n_valid, np8, agg_ref, deg_ref, ids_ref, b_ref, o_ref):
    a = agg_ref[0] + agg_ref[1]
    d = deg_ref[0] + deg_ref[1]
    norm = lax.rsqrt(jnp.maximum(d, 1.0))
    lane = lax.broadcasted_iota(jnp.int32, (np8, G), 1)
    prow = lax.broadcasted_iota(jnp.int32, (np8, G), 0)
    node = prow * 8 + lane // LANES
    hn = a * norm + b_ref[...]
    hn = hn + jnp.where(lane % LANES == 10, 1.0, 0.0)
    hn = jnp.where(node < n_valid, hn, 0.0)
    gids = lax.broadcasted_iota(jnp.int32, (np8, G), 1)
    acc = jnp.zeros((G, LANES), jnp.float32)
    for a8 in range(8):
        hn_a = lax.slice(hn, (0, a8 * LANES), (np8, (a8 + 1) * LANES))
        ids_a = ids_ref[a8]
        oh = jnp.where(ids_a[:, None] == gids, 1.0, 0.0)
        acc = acc + lax.dot_general(
            oh, hn_a, dimension_numbers=(((0,), (0,)), ((), ())),
            preferred_element_type=jnp.float32,
        )
    glane = lax.broadcasted_iota(jnp.int32, (G, LANES), 1)
    cnt = jnp.sum(jnp.where(glane == 10, acc, 0.0), axis=1, keepdims=True)
    o_ref[...] = acc / jnp.maximum(cnt, 1.0)


def kernel(in_feat, edge_index, node_graph_ids, W, b):
    n, d_in = in_feat.shape
    e = edge_index.shape[1]
    c_out = W.shape[1]

    nr = ((n + 1 + 127) // 128) * 128
    ept = e // NW
    np8 = n // 8
    nr8 = nr // 8

    w_pad = jnp.pad(W, ((0, 0), (0, LANES - c_out)))
    w_big = (jnp.eye(8, dtype=jnp.float32)[:, None, :, None]
             * w_pad[None, :, None, :]).reshape(8 * d_in, G)
    b_tile = jnp.tile(jnp.pad(b, (0, LANES - c_out)), 8).reshape(1, G)
    ids_pad = jnp.pad(node_graph_ids, (0, nr - n))
    ids_strided = ids_pad.reshape(nr8, 8).T

    deg_out, deg_in = _make_hist_kernel(n, nr, ept)(edge_index)
    deg_out_p = deg_out.reshape(NC, nr8, G)
    deg_in_p = deg_in.reshape(NC, nr8, G)

    x_packed = in_feat.reshape(np8, 8 * d_in)
    h_packed = pl.pallas_call(
        _matmul_body,
        grid=(1,),
        in_specs=[
            pl.BlockSpec((np8, 8 * d_in), lambda i: (0, 0)),
            pl.BlockSpec((8 * d_in, G), lambda i: (0, 0)),
        ],
        out_specs=pl.BlockSpec((np8, G), lambda i: (0, 0)),
        out_shape=jax.ShapeDtypeStruct((np8, G), jnp.float32),
    )(x_packed, w_big)

    h_scaled = pl.pallas_call(
        functools.partial(_scale_body, np8),
        grid=(1,),
        in_specs=[
            pl.BlockSpec((NC, nr8, G), lambda i: (0, 0, 0)),
            pl.BlockSpec((np8, G), lambda i: (0, 0)),
        ],
        out_specs=pl.BlockSpec((np8, G), lambda i: (0, 0)),
        out_shape=jax.ShapeDtypeStruct((np8, G), jnp.float32),
    )(deg_out_p, h_packed)

    agg = _make_scatter_kernel(n, nr, ept)(
        h_scaled.reshape(n, LANES), edge_index
    )
    agg_p = agg.reshape(NC, nr8, G)

    out16 = pl.pallas_call(
        functools.partial(_readout_body, n, nr8),
        grid=(1,),
        in_specs=[
            pl.BlockSpec((NC, nr8, G), lambda i: (0, 0, 0)),
            pl.BlockSpec((NC, nr8, G), lambda i: (0, 0, 0)),
            pl.BlockSpec((8, nr8), lambda i: (0, 0)),
            pl.BlockSpec((1, G), lambda i: (0, 0)),
        ],
        out_specs=pl.BlockSpec((G, LANES), lambda i: (0, 0)),
        out_shape=jax.ShapeDtypeStruct((G, LANES), jnp.float32),
    )(agg_p, deg_in_p, ids_strided, b_tile)

    return out16[:, :c_out]
